# baseline (speedup 1.0000x reference)
import functools

import jax
import jax.numpy as jnp
from jax import lax
from jax.experimental import pallas as pl
from jax.experimental.pallas import tpu as pltpu
from jax.experimental.pallas import tpu_sc as plsc

_K_RATIO = 0.1

_NBINS0 = 1024
_HISTW = 16 * _NBINS0
_SENT = 0x7FFFFFFF
_SMALL_PASSES = ((13, 8, 256), (5, 8, 256), (0, 5, 32))


def _select_pass(read_vreg, ngroups, unroll, hist_ref, kk, prefix, sh, nb,
                 nbins, lane, ones, zeros):
    def scat(i, _):
        bs = [read_vreg(i, u) for u in range(unroll)]
        idxs, ms = [], []
        for b in bs:
            hi = b >> sh
            idxs.append((((hi & jnp.int32((1 << nb) - 1)) << 4)) | lane)
            ms.append((hi >> nb) == prefix)
        for idx, m in zip(idxs, ms):
            plsc.addupdate_scatter(hist_ref, [idx], ones, mask=m)
        return 0

    lax.fori_loop(0, ngroups, scat, 0)

    def scan_b(i, st):
        carry, bcount, gtb = st
        for u in range(4):
            c = jnp.int32(nbins - 1) - (i * 4 + u)
            v = hist_ref[pl.ds(c * 16, 16)]
            hist_ref[pl.ds(c * 16, 16)] = zeros
            carry = carry + jnp.sum(v)
            bcount = bcount + (carry >= kk).astype(jnp.int32)
            gtb = jnp.maximum(gtb, jnp.where(carry < kk, carry, 0))
        return carry, bcount, gtb

    _, bcount, gtb = lax.fori_loop(
        0, nbins // 4, scan_b,
        (jnp.int32(0), jnp.int32(0), jnp.int32(0)),
    )
    return kk - gtb, (prefix << nb) | (bcount - 1)


def _row_thresh_ss(row_ref, hist_ref, cand_ref, k, n, pre_compact):
    lane = lax.iota(jnp.int32, 16)
    ones = jnp.ones((16,), jnp.int32)
    zeros = jnp.zeros((16,), jnp.int32)
    fzeros = jnp.zeros((16,), jnp.float32)
    kk = jnp.int32(k)
    prefix = jnp.int32(0)

    def read_row(i, u):
        v = row_ref[pl.ds(i * 256 + u * 16, 16)]
        return plsc.bitcast(v, jnp.int32) & jnp.int32(0x7FFFFFFF)

    kk, prefix = _select_pass(
        read_row, n // 256, 16, hist_ref, kk, prefix, 21, 10, _NBINS0,
        lane, ones, zeros
    )
    pre_compact()

    def cpt(i, st):
        off = st[0]
        accs = st[1:]
        bs = [
            plsc.bitcast(row_ref[pl.ds(i * 256 + u * 16, 16)], jnp.int32)
            & jnp.int32(0x7FFFFFFF)
            for u in range(16)
        ]
        ms = [(b >> 21) == prefix for b in bs]
        pcs = [plsc.all_reduce_population_count(m)[0] for m in ms]
        offs = []
        for pc in pcs:
            offs.append(off)
            off = off + pc
        new = list(accs)
        for u, b in enumerate(bs):
            hi = plsc.bitcast(b, jnp.float32)
            xm = jnp.where((b >> 21) > prefix, hi, 0.0)
            new[u % 8] = new[u % 8] + xm * xm
        for b, m, o in zip(bs, ms, offs):
            plsc.store_compressed(
                cand_ref.at[pl.ds(o, 16)], plsc.bitcast(b, jnp.float32),
                mask=m,
            )
        return (off, *new)

    st = lax.fori_loop(0, n // 256, cpt, (jnp.int32(0),) + (fzeros,) * 8)
    c1 = st[0]
    ss_hi = st[1] + st[2] + st[3] + st[4] + st[5] + st[6] + st[7] + st[8]
    sent = plsc.bitcast(jnp.full((16,), _SENT, jnp.int32), jnp.float32)
    for u in range(8):
        cand_ref[pl.ds(c1 + u * 16, 16)] = sent
    nit = (c1 + jnp.int32(127)) >> 7

    def read_cand(i, u):
        return plsc.bitcast(cand_ref[pl.ds(i * 128 + u * 16, 16)], jnp.int32)

    for sh, nb, nbins in _SMALL_PASSES:
        kk, prefix = _select_pass(
            read_cand, nit, 8, hist_ref, kk, prefix, sh, nb, nbins, lane,
            ones, zeros
        )

    def cssq(i, accs):
        vs = [cand_ref[pl.ds(i * 128 + u * 16, 16)] for u in range(8)]
        new = []
        for v, a in zip(vs, accs):
            b = plsc.bitcast(v, jnp.int32)
            keep = (b >= prefix) & (b < jnp.int32(_SENT))
            xm = jnp.where(keep, v, 0.0)
            new.append(a + xm * xm)
        return tuple(new)

    accs = lax.fori_loop(0, nit, cssq, (fzeros,) * 8)
    ss_cand = (accs[0] + accs[1] + accs[2] + accs[3]
               + accs[4] + accs[5] + accs[6] + accs[7])
    return prefix, jnp.sum(ss_hi + ss_cand)


def _mask_scale_row(row_ref, out_ref, tbits, ss, n):
    signmask = jnp.int32(0x7FFFFFFF)

    ssv = jnp.full((16,), ss, jnp.float32)
    y = plsc.bitcast(
        (plsc.bitcast(ssv, jnp.int32) >> 1) + jnp.int32(0x1FBD1DF5), jnp.float32
    )
    for _ in range(3):
        y = 0.5 * (y + ssv / y)
    inv = 1.0 / (y + 1e-6)
    inv = inv[0]

    def scale(i, _):
        base = i * 256
        vs = [row_ref[pl.ds(base + u * 16, 16)] for u in range(16)]
        outs = []
        for v in vs:
            b = plsc.bitcast(v, jnp.int32) & signmask
            outs.append(jnp.where(b >= tbits, v, 0.0) * inv)
        for u, o in enumerate(outs):
            out_ref[pl.ds(base + u * 16, 16)] = o
        return 0

    lax.fori_loop(0, n // 256, scale, 0)


def _make_sc_kernel(m, n, k):
    mesh = plsc.VectorSubcoreMesh(core_axis_name="c", subcore_axis_name="s")
    rows_per = m // 32

    @functools.partial(
        pl.kernel,
        mesh=mesh,
        out_type=jax.ShapeDtypeStruct((m, n), jnp.float32),
        compiler_params=pltpu.CompilerParams(needs_layout_passes=False),
        scratch_types=[
            pltpu.VMEM((n,), jnp.float32),
            pltpu.VMEM((n,), jnp.float32),
            pltpu.VMEM((_HISTW,), jnp.int32),
            pltpu.VMEM((n + 128,), jnp.float32),
            pltpu.SemaphoreType.DMA,
            pltpu.SemaphoreType.DMA,
            pltpu.SemaphoreType.DMA,
        ],
    )
    def sc_kernel(x_hbm, out_hbm, row_a, row_b, hist, cand, si_a, si_b, so):
        wid = lax.axis_index("c") * 16 + lax.axis_index("s")
        base = wid * rows_per
        bufs = (row_a, row_b)
        sin = (si_a, si_b)
        zeros = jnp.zeros((16,), jnp.int32)

        def clr(i, _):
            for u in range(8):
                hist[pl.ds(i * 128 + u * 16, 16)] = zeros
            return 0

        lax.fori_loop(0, _HISTW // 128, clr, 0)

        h_in = [None] * rows_per
        h_out = [None] * rows_per
        h_in[0] = pltpu.async_copy(x_hbm.at[base], bufs[0], sin[0])
        if rows_per > 1:
            h_in[1] = pltpu.async_copy(x_hbm.at[base + 1], bufs[1], sin[1])
        for j in range(rows_per):
            h_in[j].wait()
            buf = bufs[j % 2]
            drain = (lambda h: (lambda: h.wait()))(h_out[j - 1]) \
                if j > 0 else (lambda: None)
            t, ss = _row_thresh_ss(buf, hist, cand, k, n, drain)
            _mask_scale_row(buf, cand, t, ss, n)
            if j + 2 < rows_per:
                h_in[j + 2] = pltpu.async_copy(
                    x_hbm.at[base + j + 2], buf, sin[j % 2]
                )
            h_out[j] = pltpu.async_copy(
                cand.at[pl.ds(0, n)], out_hbm.at[base + j], so
            )
        h_out[rows_per - 1].wait()

    return sc_kernel


@jax.jit
def kernel(x):
    m, n = x.shape
    k = int(_K_RATIO * n)
    return _make_sc_kernel(m, n, k)(x)
